# baseline (device time: 35263 ns/iter reference)
import jax
import jax.numpy as jnp
from jax import lax
from jax.experimental import pallas as pl
from jax.experimental.pallas import tpu as pltpu


def kernel(Q, K, V):
    b, sq, h, d = Q.shape
    _, skv, _, _ = K.shape
    scale = d ** -0.5

    def body(q_ref, k_ref, v_ref, out_ref, send_buf, recv_buf,
             send_sem, recv_sem):
        my_x = lax.axis_index("x")
        my_y = lax.axis_index("y")
        my_z = lax.axis_index("z")
        nbr = (1 - my_x, my_y, my_z)

        barrier = pltpu.get_barrier_semaphore()
        pl.semaphore_signal(barrier, inc=1, device_id=nbr,
                            device_id_type=pl.DeviceIdType.MESH)
        pl.semaphore_wait(barrier, 1)

        q = q_ref[...]
        k = k_ref[...]
        v = v_ref[...]

        s = jnp.sum(q * k, axis=-1, keepdims=True) * scale
        p = jnp.exp(s)
        l = jnp.sum(p, axis=1)
        acc = jnp.sum(p * v, axis=1)

        send_buf[0] = acc
        send_buf[1] = jnp.broadcast_to(l, (b, h, d))

        rdma = pltpu.make_async_remote_copy(
            src_ref=send_buf,
            dst_ref=recv_buf,
            send_sem=send_sem,
            recv_sem=recv_sem,
            device_id=nbr,
            device_id_type=pl.DeviceIdType.MESH,
        )
        rdma.start()
        rdma.wait()

        acc_r = recv_buf[0]
        l_r = recv_buf[1][:, :, 0:1]
        o = (acc + acc_r) / (l + l_r)
        out_ref[...] = o.reshape(b, sq, h, d)

    return pl.pallas_call(
        body,
        out_shape=jax.ShapeDtypeStruct((b, sq, h, d), jnp.float32),
        in_specs=[pl.BlockSpec(memory_space=pltpu.VMEM)] * 3,
        out_specs=pl.BlockSpec(memory_space=pltpu.VMEM),
        scratch_shapes=[
            pltpu.VMEM((2, b, h, d), jnp.float32),
            pltpu.VMEM((2, b, h, d), jnp.float32),
            pltpu.SemaphoreType.DMA,
            pltpu.SemaphoreType.DMA,
        ],
        compiler_params=pltpu.CompilerParams(collective_id=0),
    )(Q, K, V)
